# baseline (speedup 1.0000x reference)
import functools

import jax
import jax.numpy as jnp
from jax import lax
from jax.experimental import pallas as pl
from jax.experimental.pallas import tpu as pltpu
from jax.experimental.pallas import tpu_sc as plsc

B, L, D = 1024, 200, 64
NC, NS = 2, 16
NW = NC * NS
ROWS_PER_W = B // NW
LANES = 16
VPR = D // LANES
LPAD = 208
NGRP = LPAD // LANES


def _body(x_hbm, tok_hbm, pos_hbm, out_hbm, pos_v, idx_v, buf_v, gsem):
    wid = lax.axis_index("s") * NC + lax.axis_index("c")
    row0 = wid * ROWS_PER_W

    pltpu.sync_copy(pos_hbm, pos_v)

    idx_v[pl.ds(192, LANES)] = lax.iota(jnp.int32, LANES) + wid * LANES

    def do_row(r, _):
        b = row0 + r
        pltpu.sync_copy(x_hbm.at[pl.ds(b * L, L)], idx_v.at[pl.ds(0, L)])

        def fetch(g, _):
            off = g * LANES
            vec = idx_v[pl.ds(off, LANES)]
            for k in range(LANES):
                pltpu.async_copy(tok_hbm.at[vec[k]], buf_v.at[off + k], gsem)
            return 0
        lax.fori_loop(0, NGRP, fetch, 0)
        pltpu.make_async_copy(tok_hbm.at[pl.ds(0, LPAD)], buf_v, gsem).wait()

        def add_one(i, _):
            l = i // VPR
            j = (i % VPR) * LANES
            buf_v[l, pl.ds(j, LANES)] = (
                buf_v[l, pl.ds(j, LANES)] + pos_v[pl.ds(l * D + j, LANES)])
            return 0
        lax.fori_loop(0, L * VPR, add_one, 0, unroll=8)

        pltpu.sync_copy(buf_v.at[pl.ds(0, L)], out_hbm.at[b])
        return 0

    lax.fori_loop(0, ROWS_PER_W, do_row, 0)


def kernel(x, token_table, pos_table):
    mesh = plsc.VectorSubcoreMesh(
        core_axis_name="c", subcore_axis_name="s",
        num_cores=NC, num_subcores=NS)
    run = pl.kernel(
        _body,
        out_type=jax.ShapeDtypeStruct((B, L, D), jnp.float32),
        mesh=mesh,
        scratch_types=[
            pltpu.VMEM((L * D,), jnp.float32),
            pltpu.VMEM((LPAD,), jnp.int32),
            pltpu.VMEM((LPAD, D), jnp.float32),
            pltpu.SemaphoreType.DMA,
        ],
    )
    x_flat = x.astype(jnp.int32).reshape(-1)
    pos_flat = pos_table.reshape(-1)
    return run(x_flat, token_table, pos_flat)


# speedup vs baseline: 1.0882x; 1.0882x over previous
import functools

import jax
import jax.numpy as jnp
from jax import lax
from jax.experimental import pallas as pl
from jax.experimental.pallas import tpu as pltpu
from jax.experimental.pallas import tpu_sc as plsc

B, L, D = 1024, 200, 64
NC, NS = 2, 16
NW = NC * NS
ROWS_PER_W = B // NW
TOK_PER_W = ROWS_PER_W * L
LANES = 16
VPR = D // LANES
LPAD = 208
NGRP = LPAD // LANES
NPAIR = ROWS_PER_W // 2


def _body(x_hbm, tok_hbm, pos_hbm, out_hbm,
          pos_v, idx_v, buf_v, isem, gsem0, gsem1, osem0, osem1):
    wid = lax.axis_index("s") * NC + lax.axis_index("c")
    row0 = wid * ROWS_PER_W

    gsems = (gsem0, gsem1)
    osems = (osem0, osem1)

    pltpu.async_copy(x_hbm.at[pl.ds(row0 * L, TOK_PER_W)],
                     idx_v.at[pl.ds(0, TOK_PER_W)], isem)
    pltpu.sync_copy(pos_hbm, pos_v)
    idx_v[pl.ds(TOK_PER_W, LANES)] = lax.iota(jnp.int32, LANES) + wid * LANES
    pltpu.make_async_copy(x_hbm.at[pl.ds(row0 * L, TOK_PER_W)],
                          idx_v.at[pl.ds(0, TOK_PER_W)], isem).wait()

    def enqueue_gathers(s, r):
        buf_s = buf_v.at[s]
        base = r * L

        def fetch(g, _):
            off = g * LANES
            vec = idx_v[pl.ds(base + off, LANES)]
            for k in range(LANES):
                pltpu.async_copy(tok_hbm.at[vec[k]], buf_s.at[off + k],
                                 gsems[s])
            return 0
        lax.fori_loop(0, NGRP, fetch, 0)

    def finish_row(s, b):
        buf_s = buf_v.at[s]
        pltpu.make_async_copy(tok_hbm.at[pl.ds(0, LPAD)], buf_s,
                              gsems[s]).wait()

        def add_one(i, _):
            l = i // VPR
            j = (i % VPR) * LANES
            buf_s[l, pl.ds(j, LANES)] = (
                buf_s[l, pl.ds(j, LANES)] + pos_v[pl.ds(l * D + j, LANES)])
            return 0
        lax.fori_loop(0, L * VPR, add_one, 0, unroll=8)

        pltpu.async_copy(buf_s.at[pl.ds(0, L)], out_hbm.at[b], osems[s])

    def out_wait(s, b):
        pltpu.make_async_copy(buf_v.at[s].at[pl.ds(0, L)], out_hbm.at[b],
                              osems[s]).wait()

    def pair(g, _):
        @pl.when(g > 0)
        def _():
            out_wait(0, row0 + 2 * g - 2)
        enqueue_gathers(0, 2 * g)

        @pl.when(g > 0)
        def _():
            finish_row(1, row0 + 2 * g - 1)

        @pl.when(g > 0)
        def _():
            out_wait(1, row0 + 2 * g - 1)
        enqueue_gathers(1, 2 * g + 1)

        finish_row(0, row0 + 2 * g)
        return 0

    lax.fori_loop(0, NPAIR, pair, 0)

    last1 = row0 + ROWS_PER_W - 1
    finish_row(1, last1)
    out_wait(0, last1 - 1)
    out_wait(1, last1)


def kernel(x, token_table, pos_table):
    mesh = plsc.VectorSubcoreMesh(
        core_axis_name="c", subcore_axis_name="s",
        num_cores=NC, num_subcores=NS)
    run = pl.kernel(
        _body,
        out_type=jax.ShapeDtypeStruct((B, L, D), jnp.float32),
        mesh=mesh,
        scratch_types=[
            pltpu.VMEM((L * D,), jnp.float32),
            pltpu.VMEM((TOK_PER_W + LANES,), jnp.int32),
            pltpu.VMEM((2, LPAD, D), jnp.float32),
            pltpu.SemaphoreType.DMA,
            pltpu.SemaphoreType.DMA,
            pltpu.SemaphoreType.DMA,
            pltpu.SemaphoreType.DMA,
            pltpu.SemaphoreType.DMA,
        ],
    )
    x_flat = x.astype(jnp.int32).reshape(-1)
    pos_flat = pos_table.reshape(-1)
    return run(x_flat, token_table, pos_flat)
